# baseline (speedup 1.0000x reference)
import functools

import jax
import jax.numpy as jnp
from jax import lax
from jax.experimental import pallas as pl
from jax.experimental.pallas import tpu as pltpu
from jax.experimental.pallas import tpu_sc as plsc

_VOCAB = 1000000
_D = 64
_CLS = 4
_LANES = 16
_BATCH = 16384
_HIST = 200
_NCORES = 2
_NSUB = 16
_NWORK = _NCORES * _NSUB
_BAGS_PER_WORK = _BATCH // _NWORK
_LCHUNK = 4
_CHUNKS = _HIST // _LCHUNK
_JSPLIT = _BAGS_PER_WORK // 128
_ROWS_PER_CHUNK = _LCHUNK * _BAGS_PER_WORK


_FBLK = 32768


def _fold_body(t_ref, w_ref, p_ref, s_ref):
    s_ref[...] = lax.dot_general(
        t_ref[...].astype(jnp.bfloat16), w_ref[...].astype(jnp.bfloat16),
        dimension_numbers=(((0,), (0,)), ((), ())),
        preferred_element_type=jnp.float32)
    for u in range(8):
        p_ref[:, u * _LANES:(u + 1) * _LANES] = (
            s_ref[pl.ds(u, _FBLK // 8, 8), :])


def _fold_table(table_t, Wp):
    return pl.pallas_call(
        _fold_body,
        grid=(pl.cdiv(_VOCAB, _FBLK),),
        in_specs=[
            pl.BlockSpec((_D, _FBLK), lambda i: (0, i)),
            pl.BlockSpec((_D, _LANES), lambda i: (0, 0)),
        ],
        out_specs=pl.BlockSpec((_FBLK // 8, 8 * _LANES), lambda i: (i, 0)),
        out_shape=jax.ShapeDtypeStruct((_VOCAB // 8, 8 * _LANES), jnp.float32),
        scratch_shapes=[pltpu.VMEM((_FBLK, _LANES), jnp.float32)],
    )(table_t, Wp)


def _gather_descs(p_hbm, idx_buf, rows_buf, sem):
    descs = []
    for l in range(_LCHUNK):
        for j in range(_JSPLIT):
            descs.append(pltpu.make_async_copy(
                p_hbm.at[idx_buf.at[l, pl.ds(j * 128, 128)]],
                rows_buf.at[pl.ds(l * _BAGS_PER_WORK + j * 128, 128)], sem))
    return descs


def _fire_gathers(p_hbm, idx_buf, rows_buf, sem):
    for d in _gather_descs(p_hbm, idx_buf, rows_buf, sem):
        d.start()


def _wait_gathers(p_hbm, idx_buf, rows_buf, sem):
    for d in _gather_descs(p_hbm, idx_buf, rows_buf, sem):
        d.wait()


def _reduce_chunk(rows_buf, acc_v):
    @pl.loop(0, _BAGS_PER_WORK)
    def _(c):
        r01 = rows_buf[c] + rows_buf[_BAGS_PER_WORK + c]
        r23 = (rows_buf[2 * _BAGS_PER_WORK + c]
               + rows_buf[3 * _BAGS_PER_WORK + c])
        acc_v[c] = acc_v[c] + (r01 + r23)


_sc_mesh = plsc.VectorSubcoreMesh(core_axis_name="c", subcore_axis_name="s")


@functools.partial(
    pl.kernel,
    out_type=jax.ShapeDtypeStruct((_BATCH, _LANES), jnp.float32),
    mesh=_sc_mesh,
    compiler_params=pltpu.CompilerParams(use_tc_tiling_on_sc=False),
    scratch_types=[
        pltpu.VMEM((2, _LCHUNK, _BAGS_PER_WORK), jnp.int32),
        pltpu.VMEM((2, _ROWS_PER_CHUNK, _LANES), jnp.float32),
        pltpu.VMEM((_BAGS_PER_WORK, _LANES), jnp.float32),
        pltpu.SemaphoreType.DMA,
        pltpu.SemaphoreType.DMA,
        pltpu.SemaphoreType.DMA,
        pltpu.SemaphoreType.DMA,
    ],
)
def _sc_embed(p_hbm, idx_hbm, out_hbm, idx_v, rows_v, acc_v,
              gsem0, gsem1, isem0, isem1):
    wid = lax.axis_index("c") * _NSUB + lax.axis_index("s")
    bag0 = wid * _BAGS_PER_WORK
    gsems = (gsem0, gsem1)
    isems = (isem0, isem1)

    @pl.loop(0, _BAGS_PER_WORK)
    def _(c):
        acc_v[c] = jnp.zeros((_LANES,), jnp.float32)

    pltpu.sync_copy(
        idx_hbm.at[pl.ds(0, _LCHUNK), pl.ds(bag0, _BAGS_PER_WORK)],
        idx_v.at[0])
    _fire_gathers(p_hbm, idx_v.at[0], rows_v.at[0], gsem0)
    pltpu.make_async_copy(
        idx_hbm.at[pl.ds(_LCHUNK, _LCHUNK), pl.ds(bag0, _BAGS_PER_WORK)],
        idx_v.at[1], isem1,
    ).start()

    @pl.loop(0, _CHUNKS // 2)
    def _(g):
        for par in (0, 1):
            ch = g * 2 + par
            q = 1 - par

            @pl.when(ch < _CHUNKS - 1)
            def _():
                pltpu.make_async_copy(
                    idx_hbm.at[pl.ds((ch + 1) * _LCHUNK, _LCHUNK),
                               pl.ds(bag0, _BAGS_PER_WORK)],
                    idx_v.at[q], isems[q],
                ).wait()
                _fire_gathers(p_hbm, idx_v.at[q], rows_v.at[q], gsems[q])

            _wait_gathers(p_hbm, idx_v.at[par], rows_v.at[par], gsems[par])

            @pl.when(ch < _CHUNKS - 2)
            def _():
                pltpu.make_async_copy(
                    idx_hbm.at[pl.ds((ch + 2) * _LCHUNK, _LCHUNK),
                               pl.ds(bag0, _BAGS_PER_WORK)],
                    idx_v.at[par], isems[par],
                ).start()

            _reduce_chunk(rows_v.at[par], acc_v)

    pltpu.sync_copy(acc_v, out_hbm.at[pl.ds(bag0, _BAGS_PER_WORK)])


def kernel(text, table, W, b):
    Wp = jnp.zeros((_D, _LANES), jnp.float32)
    Wp = Wp.at[:, :_CLS].set(W.T * (1.0 / _HIST))
    P = _fold_table(table.T, Wp).reshape(_VOCAB, _LANES)
    pooled = _sc_embed(P, text.astype(jnp.int32).T)
    return pooled[:, :_CLS] + b


# speedup vs baseline: 1.0182x; 1.0182x over previous
import functools

import jax
import jax.numpy as jnp
from jax import lax
from jax.experimental import pallas as pl
from jax.experimental.pallas import tpu as pltpu
from jax.experimental.pallas import tpu_sc as plsc

_VOCAB = 1000000
_D = 64
_CLS = 4
_LANES = 16
_BATCH = 16384
_HIST = 200
_NCORES = 2
_NSUB = 16
_NWORK = _NCORES * _NSUB
_BAGS_PER_WORK = _BATCH // _NWORK
_LCHUNK = 5
_CHUNKS = _HIST // _LCHUNK
_JSPLIT = _BAGS_PER_WORK // 128
_ROWS_PER_CHUNK = _LCHUNK * _BAGS_PER_WORK


_FBLK = 32768


def _fold_body(t_ref, w_ref, p_ref, s_ref):
    s_ref[...] = lax.dot_general(
        t_ref[...].astype(jnp.bfloat16), w_ref[...].astype(jnp.bfloat16),
        dimension_numbers=(((0,), (0,)), ((), ())),
        preferred_element_type=jnp.float32)
    for u in range(8):
        p_ref[:, u * _LANES:(u + 1) * _LANES] = (
            s_ref[pl.ds(u, _FBLK // 8, 8), :])


def _fold_table(table_t, Wp):
    return pl.pallas_call(
        _fold_body,
        grid=(pl.cdiv(_VOCAB, _FBLK),),
        in_specs=[
            pl.BlockSpec((_D, _FBLK), lambda i: (0, i)),
            pl.BlockSpec((_D, _LANES), lambda i: (0, 0)),
        ],
        out_specs=pl.BlockSpec((_FBLK // 8, 8 * _LANES), lambda i: (i, 0)),
        out_shape=jax.ShapeDtypeStruct((_VOCAB // 8, 8 * _LANES), jnp.float32),
        scratch_shapes=[pltpu.VMEM((_FBLK, _LANES), jnp.float32)],
    )(table_t, Wp)


def _gather_descs(p_hbm, idx_buf, rows_buf, sem):
    descs = []
    for l in range(_LCHUNK):
        for j in range(_JSPLIT):
            descs.append(pltpu.make_async_copy(
                p_hbm.at[idx_buf.at[l, pl.ds(j * 128, 128)]],
                rows_buf.at[pl.ds(l * _BAGS_PER_WORK + j * 128, 128)], sem))
    return descs


def _fire_gathers(p_hbm, idx_buf, rows_buf, sem):
    for d in _gather_descs(p_hbm, idx_buf, rows_buf, sem):
        d.start()


def _wait_gathers(p_hbm, idx_buf, rows_buf, sem):
    for d in _gather_descs(p_hbm, idx_buf, rows_buf, sem):
        d.wait()


def _reduce_chunk(rows_buf, acc_v):
    @pl.loop(0, _BAGS_PER_WORK)
    def _(c):
        terms = [rows_buf[l * _BAGS_PER_WORK + c] for l in range(_LCHUNK)]
        while len(terms) > 1:
            terms = [terms[i] + terms[i + 1] if i + 1 < len(terms)
                     else terms[i] for i in range(0, len(terms), 2)]
        acc_v[c] = acc_v[c] + terms[0]


_sc_mesh = plsc.VectorSubcoreMesh(core_axis_name="c", subcore_axis_name="s")


@functools.partial(
    pl.kernel,
    out_type=jax.ShapeDtypeStruct((_BATCH, _LANES), jnp.float32),
    mesh=_sc_mesh,
    compiler_params=pltpu.CompilerParams(use_tc_tiling_on_sc=False),
    scratch_types=[
        pltpu.VMEM((2, _LCHUNK, _BAGS_PER_WORK), jnp.int32),
        pltpu.VMEM((2, _ROWS_PER_CHUNK, _LANES), jnp.float32),
        pltpu.VMEM((_BAGS_PER_WORK, _LANES), jnp.float32),
        pltpu.SemaphoreType.DMA,
        pltpu.SemaphoreType.DMA,
        pltpu.SemaphoreType.DMA,
        pltpu.SemaphoreType.DMA,
    ],
)
def _sc_embed(p_hbm, idx_hbm, out_hbm, idx_v, rows_v, acc_v,
              gsem0, gsem1, isem0, isem1):
    wid = lax.axis_index("c") * _NSUB + lax.axis_index("s")
    bag0 = wid * _BAGS_PER_WORK
    gsems = (gsem0, gsem1)
    isems = (isem0, isem1)

    @pl.loop(0, _BAGS_PER_WORK)
    def _(c):
        acc_v[c] = jnp.zeros((_LANES,), jnp.float32)

    pltpu.sync_copy(
        idx_hbm.at[pl.ds(0, _LCHUNK), pl.ds(bag0, _BAGS_PER_WORK)],
        idx_v.at[0])
    _fire_gathers(p_hbm, idx_v.at[0], rows_v.at[0], gsem0)
    pltpu.make_async_copy(
        idx_hbm.at[pl.ds(_LCHUNK, _LCHUNK), pl.ds(bag0, _BAGS_PER_WORK)],
        idx_v.at[1], isem1,
    ).start()

    @pl.loop(0, _CHUNKS // 2)
    def _(g):
        for par in (0, 1):
            ch = g * 2 + par
            q = 1 - par

            @pl.when(ch < _CHUNKS - 1)
            def _():
                pltpu.make_async_copy(
                    idx_hbm.at[pl.ds((ch + 1) * _LCHUNK, _LCHUNK),
                               pl.ds(bag0, _BAGS_PER_WORK)],
                    idx_v.at[q], isems[q],
                ).wait()
                _fire_gathers(p_hbm, idx_v.at[q], rows_v.at[q], gsems[q])

            _wait_gathers(p_hbm, idx_v.at[par], rows_v.at[par], gsems[par])

            @pl.when(ch < _CHUNKS - 2)
            def _():
                pltpu.make_async_copy(
                    idx_hbm.at[pl.ds((ch + 2) * _LCHUNK, _LCHUNK),
                               pl.ds(bag0, _BAGS_PER_WORK)],
                    idx_v.at[par], isems[par],
                ).start()

            _reduce_chunk(rows_v.at[par], acc_v)

    pltpu.sync_copy(acc_v, out_hbm.at[pl.ds(bag0, _BAGS_PER_WORK)])


def kernel(text, table, W, b):
    Wp = jnp.zeros((_D, _LANES), jnp.float32)
    Wp = Wp.at[:, :_CLS].set(W.T * (1.0 / _HIST))
    P = _fold_table(table.T, Wp).reshape(_VOCAB, _LANES)
    pooled = _sc_embed(P, text.astype(jnp.int32).T)
    return pooled[:, :_CLS] + b
